# baseline (speedup 1.0000x reference)
import functools

import jax
import jax.numpy as jnp
from jax.experimental import pallas as pl

NUM_GRAPHS = 256


def _edge_mlp_body(ea_ref, w1_ref, b1_ref, w2_ref, b2_ref, out_ref):
    eb = ea_ref[...]
    a = jnp.maximum(jnp.dot(eb, w1_ref[...],
                            preferred_element_type=jnp.float32) + b1_ref[...], 0.0)
    o = (a * w2_ref[...]).sum(axis=1, keepdims=True) + b2_ref[...]
    out_ref[...] = jnp.maximum(o, 0.0)


@functools.partial(jax.jit, static_argnames=())
def _edge_mlp(edge_attr, d1W, d1b, d2W, d2b):
    E, F = edge_attr.shape
    bm = 8000
    grid = (E // bm,)
    out = pl.pallas_call(
        _edge_mlp_body,
        grid=grid,
        in_specs=[
            pl.BlockSpec((bm, F), lambda i: (i, 0)),
            pl.BlockSpec((F, 64), lambda i: (0, 0)),
            pl.BlockSpec((1, 64), lambda i: (0, 0)),
            pl.BlockSpec((1, 64), lambda i: (0, 0)),
            pl.BlockSpec((1, 1), lambda i: (0, 0)),
        ],
        out_specs=pl.BlockSpec((bm, 1), lambda i: (i, 0)),
        out_shape=jax.ShapeDtypeStruct((E, 1), jnp.float32),
    )(edge_attr, d1W, d1b.reshape(1, 64), d2W.reshape(1, 64), d2b.reshape(1, 1))
    return out


def _segsum(data, ids, n):
    return jax.ops.segment_sum(data, ids, num_segments=n)


def _topk_stage(x, w, batch, node_mask, ratio=0.8):
    N = x.shape[0]
    score = (x * w).sum(-1) / (jnp.linalg.norm(w) + 1e-16)
    s = jnp.where(node_mask, jax.nn.sigmoid(score), -1.0)
    key = batch.astype(jnp.float32) * 4.0 - s
    order = jnp.argsort(key)
    rank = jnp.zeros(N, jnp.int32).at[order].set(jnp.arange(N, dtype=jnp.int32))
    counts = _segsum(jnp.ones(N, jnp.float32), batch, NUM_GRAPHS)
    starts = (jnp.cumsum(counts) - counts).astype(jnp.int32)
    rig = rank - starts[batch]
    nvalid = _segsum(node_mask.astype(jnp.float32), batch, NUM_GRAPHS)
    k = jnp.ceil(ratio * nvalid).astype(jnp.int32)
    keep = node_mask & (rig < k[batch])
    xn = x * jnp.tanh(score)[:, None] * keep.astype(x.dtype)[:, None]
    return xn, keep


def _pools_stage(x, batch, mask):
    m = mask.astype(x.dtype)[:, None]
    summ = _segsum(x * m, batch, NUM_GRAPHS)
    cnt = _segsum(m[:, 0], batch, NUM_GRAPHS)
    gap = summ / jnp.maximum(cnt, 1.0)[:, None]
    xm = jnp.where(mask[:, None], x, -1e30)
    gmp = jax.ops.segment_max(xm, batch, num_segments=NUM_GRAPHS)
    gmp = jnp.where(cnt[:, None] > 0, gmp, 0.0)
    return jnp.concatenate([gmp, gap], axis=1)


def _gcn_stage(x, W, b, src, dst, emask, nmask, N):
    em = emask.astype(jnp.float32)
    nm = nmask.astype(jnp.float32)
    deg = _segsum(em, dst, N) + nm
    dinv = jnp.where(deg > 0, 1.0 / jnp.sqrt(deg), 0.0)
    xw = x @ W
    coef = dinv[src] * dinv[dst] * em
    out = _segsum(xw[src] * coef[:, None], dst, N)
    out = out + xw * (dinv * dinv * nm)[:, None]
    return (out + b) * nm[:, None]


def _gat_stage(x, W, a_s, a_d, b, src, dst, emask, nmask, N):
    xw = x @ W
    asrc = (xw * a_s).sum(-1)
    adst = (xw * a_d).sum(-1)
    loop = jnp.arange(N)
    es = jnp.concatenate([src, loop])
    ed = jnp.concatenate([dst, loop])
    em = jnp.concatenate([emask, nmask])
    logit = jax.nn.leaky_relu(asrc[es] + adst[ed], 0.2)
    logit = jnp.where(em, logit, -1e30)
    mx = jax.ops.segment_max(logit, ed, num_segments=N)
    mx = jnp.where(mx > -1e29, mx, 0.0)
    ex = jnp.exp(logit - mx[ed]) * em.astype(jnp.float32)
    den = _segsum(ex, ed, N)
    alpha = ex / jnp.maximum(den[ed], 1e-16)
    out = _segsum(xw[es] * alpha[:, None], ed, N)
    return (out + b) * nmask.astype(jnp.float32)[:, None]


def kernel(x, edge_index, edge_attr, batch, d1W, d1b, d2W, d2b, lW, lb, p1w,
           gW, gb, p2w, gaW, gas, gad, gab, p3w, l1W, l1b, l2W, l2b, l3W, l3b):
    N = x.shape[0]
    src = edge_index[0]
    dst = edge_index[1]

    ea = _edge_mlp(edge_attr, d1W, d1b, d2W, d2b)

    msg = jax.nn.relu(x[src] + ea)
    aggr = _segsum(msg, dst, N)
    h = jax.nn.relu((x + aggr) @ lW + lb)

    nmask = jnp.ones(N, bool)
    emask = jnp.ones(src.shape[0], bool)

    h, keep = _topk_stage(h, p1w, batch, nmask)
    nmask = keep
    emask = emask & keep[src] & keep[dst]
    x1 = _pools_stage(h, batch, nmask)

    h = jax.nn.relu(_gcn_stage(h, gW, gb, src, dst, emask, nmask, N))
    h, keep = _topk_stage(h, p2w, batch, nmask)
    nmask = keep
    emask = emask & keep[src] & keep[dst]
    x2 = _pools_stage(h, batch, nmask)

    h = jax.nn.relu(_gat_stage(h, gaW, gas, gad, gab, src, dst, emask, nmask, N))
    h, keep = _topk_stage(h, p3w, batch, nmask)
    nmask = keep
    x3 = _pools_stage(h, batch, nmask)

    z = x1 + x2 + x3
    z = jax.nn.relu(z @ l1W + l1b)
    z = jax.nn.relu(z @ l2W + l2b)
    z = z @ l3W + l3b
    return jax.nn.log_softmax(z, axis=-1)
